# Initial kernel scaffold; baseline (speedup 1.0000x reference)
import jax
import jax.numpy as jnp
from jax.experimental import pallas as pl


def kernel(xyz, tables, W0, b0, W1, b1, W2, b2):
    raise NotImplementedError("write your pallas kernel here")


# speedup vs baseline: 31.8273x; 31.8273x over previous
import functools

import jax
import jax.numpy as jnp
import numpy as np
from jax import lax
from jax.experimental import pallas as pl
from jax.experimental.pallas import tpu as pltpu
from jax.experimental.pallas import tpu_sc as plsc

N = 524288
N_LEVELS = 16
F = 2
LOG2_T = 19
TABLE_SIZE = 1 << LOG2_T
BASE_RES = 16
SCALE = 1.38
WIDTH = 64
OUT_DIM = 16
IN_DIM = N_LEVELS * F

NC = 2
NS = 16
L = 16
NW = NC * NS
PW = N // NW
B = 64
CH = PW // B
RPP = N_LEVELS * 8
R = B * RPP

P1 = 2654435761 - (1 << 32)
P2 = 805459861
MASK = TABLE_SIZE - 1

RES = [float(np.floor(BASE_RES * SCALE**l)) for l in range(N_LEVELS)]

_mesh = plsc.VectorSubcoreMesh(core_axis_name="c", subcore_axis_name="s")


@functools.partial(
    pl.kernel,
    out_type=jax.ShapeDtypeStruct((IN_DIM, N), jnp.float32),
    mesh=_mesh,
    compiler_params=pltpu.CompilerParams(
        use_tc_tiling_on_sc=False, needs_layout_passes=False
    ),
    scratch_types=[
        pltpu.VMEM((B,), jnp.float32),
        pltpu.VMEM((B,), jnp.float32),
        pltpu.VMEM((B,), jnp.float32),
        pltpu.VMEM((R,), jnp.int32),
        pltpu.VMEM((R, F), jnp.float32),
        pltpu.VMEM((IN_DIM * B,), jnp.float32),
        pltpu.SemaphoreType.DMA,
    ],
)
def _encode(xs, ys, zs, tab, enc, xv, yv, zv, idxv, rowsv, outv, sem):
    wid = lax.axis_index("s") * NC + lax.axis_index("c")
    iota = lax.iota(jnp.int32, L)

    def chunk_body(ch, carry):
        base = wid * PW + ch * B
        pltpu.sync_copy(xs.at[pl.ds(base, B)], xv)
        pltpu.sync_copy(ys.at[pl.ds(base, B)], yv)
        pltpu.sync_copy(zs.at[pl.ds(base, B)], zv)

        def phase_a(v, c2):
            x = xv[pl.ds(v * L, L)]
            y = yv[pl.ds(v * L, L)]
            z = zv[pl.ds(v * L, L)]
            for l in range(N_LEVELS):
                res = RES[l]
                px = (x * res).astype(jnp.int32)
                py = (y * res).astype(jnp.int32)
                pz = (z * res).astype(jnp.int32)
                loff = l * TABLE_SIZE
                hyt = py * P1
                hy = (jnp.bitwise_and(hyt, MASK), jnp.bitwise_and(hyt + P1, MASK))
                hzt = pz * P2
                hz = (
                    jnp.bitwise_or(jnp.bitwise_and(hzt, MASK), loff),
                    jnp.bitwise_or(jnp.bitwise_and(hzt + P2, MASK), loff),
                )
                hxy = (px ^ hy[0], px ^ hy[1], (px + 1) ^ hy[0], (px + 1) ^ hy[1])
                for c in range(8):
                    bx, by, bz = (c >> 2) & 1, (c >> 1) & 1, c & 1
                    h = hxy[bx * 2 + by] ^ hz[bz]
                    idxv[pl.ds((l * 8 + c) * B + v * L, L)] = h
            return c2

        lax.fori_loop(0, B // L, phase_a, 0)

        pltpu.async_copy(tab.at[idxv], rowsv, sem).wait()

        def phase_b(v, c2):
            pids = v * L + iota
            x = xv[pl.ds(v * L, L)]
            y = yv[pl.ds(v * L, L)]
            z = zv[pl.ds(v * L, L)]
            zero = jnp.zeros((L,), jnp.int32)
            one = jnp.full((L,), 1, jnp.int32)
            for l in range(N_LEVELS):
                res = RES[l]
                posx, posy, posz = x * res, y * res, z * res
                px = posx.astype(jnp.int32)
                py = posy.astype(jnp.int32)
                pz = posz.astype(jnp.int32)
                fx = posx - px.astype(jnp.float32)
                fy = posy - py.astype(jnp.float32)
                fz = posz - pz.astype(jnp.float32)
                wx = (1.0 - fx, fx)
                wy = (1.0 - fy, fy)
                wz = (1.0 - fz, fz)
                wyz = (wy[0] * wz[0], wy[0] * wz[1], wy[1] * wz[0], wy[1] * wz[1])
                acc0 = jnp.zeros((L,), jnp.float32)
                acc1 = jnp.zeros((L,), jnp.float32)
                for c in range(8):
                    bx, by, bz = (c >> 2) & 1, (c >> 1) & 1, c & 1
                    w = wx[bx] * wyz[by * 2 + bz]
                    rows = (l * 8 + c) * B + pids
                    f0 = plsc.load_gather(rowsv, [rows, zero])
                    f1 = plsc.load_gather(rowsv, [rows, one])
                    acc0 = acc0 + f0 * w
                    acc1 = acc1 + f1 * w
                outv[pl.ds((2 * l) * B + v * L, L)] = acc0
                outv[pl.ds((2 * l + 1) * B + v * L, L)] = acc1
            return c2

        lax.fori_loop(0, B // L, phase_b, 0)

        for f in range(IN_DIM):
            pltpu.sync_copy(
                outv.at[pl.ds(f * B, B)], enc.at[f, pl.ds(base, B)]
            )
        return carry

    lax.fori_loop(0, CH, chunk_body, 0)


BN = 4096


def _mlp_body(encT_ref, w0, b0, w1, b1, w2, b2, out_ref):
    dn = (((0,), (0,)), ((), ()))
    h = jnp.maximum(
        lax.dot_general(encT_ref[...], w0[...], dn, preferred_element_type=jnp.float32)
        + b0[...],
        0.0,
    )
    h = jnp.maximum(
        jnp.dot(h, w1[...], preferred_element_type=jnp.float32) + b1[...], 0.0
    )
    out_ref[...] = jnp.dot(h, w2[...], preferred_element_type=jnp.float32) + b2[...]


def _mlp(encT, W0, b0, W1, b1, W2, b2):
    return pl.pallas_call(
        _mlp_body,
        grid=(N // BN,),
        in_specs=[
            pl.BlockSpec((IN_DIM, BN), lambda i: (0, i)),
            pl.BlockSpec((IN_DIM, WIDTH), lambda i: (0, 0)),
            pl.BlockSpec((1, WIDTH), lambda i: (0, 0)),
            pl.BlockSpec((WIDTH, WIDTH), lambda i: (0, 0)),
            pl.BlockSpec((1, WIDTH), lambda i: (0, 0)),
            pl.BlockSpec((WIDTH, OUT_DIM), lambda i: (0, 0)),
            pl.BlockSpec((1, OUT_DIM), lambda i: (0, 0)),
        ],
        out_specs=pl.BlockSpec((BN, OUT_DIM), lambda i: (i, 0)),
        out_shape=jax.ShapeDtypeStruct((N, OUT_DIM), jnp.float32),
    )(
        encT,
        W0,
        b0.reshape(1, WIDTH),
        W1,
        b1.reshape(1, WIDTH),
        W2,
        b2.reshape(1, OUT_DIM),
    )


def kernel(xyz, tables, W0, b0, W1, b1, W2, b2):
    tab = tables.reshape(N_LEVELS * TABLE_SIZE, F)
    encT = _encode(xyz[:, 0], xyz[:, 1], xyz[:, 2], tab)
    return _mlp(encT, W0, b0, W1, b1, W2, b2)
